# baseline (speedup 1.0000x reference)
import functools

import jax
import jax.numpy as jnp
from jax import lax
from jax.experimental import pallas as pl
from jax.experimental.pallas import tpu as pltpu
from jax.experimental.pallas import tpu_sc as plsc

NC = 2
NS = 16
LANES = 16
CHUNK = 128
NBUF = 4
ROWS = 1000
EPS = 1e-5


def _sc_degree(edges3, zeros8, ones8, n_acc, e_work):
    e_per_tile = e_work // (NC * NS)
    n_rounds = e_per_tile // (CHUNK * NBUF)
    cpt = e_per_tile // CHUNK
    nrt = n_acc // NS
    mesh = plsc.VectorSubcoreMesh(core_axis_name="c", subcore_axis_name="s")

    @functools.partial(
        pl.kernel,
        out_type=jax.ShapeDtypeStruct((NC, n_acc, 8), jnp.float32),
        mesh=mesh,
        compiler_params=pltpu.CompilerParams(use_tc_tiling_on_sc=False),
        scratch_types=[
            pltpu.VMEM((2, NBUF, CHUNK), jnp.int32),
            pltpu.VMEM((CHUNK, 8), jnp.float32),
            pltpu.VMEM_SHARED((n_acc, 8), jnp.float32),
        ] + [pltpu.SemaphoreType.DMA] * NBUF,
    )
    def deg_kernel(edges_hbm, zeros_hbm, ones_hbm, out_hbm, didx, ones_v,
                   acc, *ssem):
        c = lax.axis_index("c")
        s = lax.axis_index("s")
        pltpu.sync_copy(zeros_hbm.at[pl.ds(s * nrt, nrt)],
                        acc.at[pl.ds(s * nrt, nrt)])
        pltpu.sync_copy(ones_hbm, ones_v)
        plsc.subcore_barrier()
        cb = (c * NS + s) * cpt

        def idx_load(r, p):
            pltpu.sync_copy(edges_hbm.at[1, pl.ds(cb + r * NBUF, NBUF)],
                            didx.at[p])

        def scatter_start(p, b):
            pltpu.async_copy(ones_v, acc.at[didx.at[p, b]], ssem[b],
                             add=True)

        def scatter_wait(p, b):
            pltpu.make_async_copy(ones_v, acc.at[didx.at[p, b]],
                                  ssem[b]).wait()

        idx_load(0, 0)

        @pl.loop(0, n_rounds, step=2)
        def _(r0):
            for p in range(2):
                idx_load(r0 + p + 1, 1 - p)
                for b in range(NBUF):
                    scatter_start(p, b)
                for b in range(NBUF):
                    scatter_wait(p, b)

        plsc.subcore_barrier()
        pltpu.sync_copy(acc.at[pl.ds(s * nrt, nrt)],
                        out_hbm.at[c, pl.ds(s * nrt, nrt)])

    return deg_kernel(edges3, zeros8, ones8)


def _sc_aggregate(xs_flat, edges3, n_acc, half, e_work):
    e_per_tile = e_work // NS
    n_rounds = e_per_tile // (CHUNK * NBUF)
    cpt = e_per_tile // CHUNK
    nrt = n_acc // NS
    mesh = plsc.VectorSubcoreMesh(core_axis_name="c", subcore_axis_name="s")

    @functools.partial(
        pl.kernel,
        out_type=jax.ShapeDtypeStruct((NC, n_acc, half), jnp.float32),
        mesh=mesh,
        compiler_params=pltpu.CompilerParams(use_tc_tiling_on_sc=False),
        scratch_types=[
            pltpu.VMEM((2, 2, NBUF, CHUNK), jnp.int32),
            pltpu.VMEM((NBUF, CHUNK, half), jnp.float32),
            pltpu.VMEM_SHARED((n_acc, half), jnp.float32),
        ] + [pltpu.SemaphoreType.DMA] * (2 * NBUF),
    )
    def agg_kernel(xs_hbm, edges_hbm, out_hbm, eidx, rows, acc, *sems):
        gsem, ssem = sems[:NBUF], sems[NBUF:]
        c = lax.axis_index("c")
        s = lax.axis_index("s")
        coff = c * n_acc
        pltpu.sync_copy(xs_hbm.at[pl.ds(coff + s * nrt, nrt)],
                        acc.at[pl.ds(s * nrt, nrt)])
        plsc.subcore_barrier()
        cb = s * cpt

        def idx_load(r, p):
            pltpu.sync_copy(edges_hbm.at[:, pl.ds(cb + r * NBUF, NBUF)],
                            eidx.at[p])
            for b in range(NBUF):
                for v in range(CHUNK // LANES):
                    sl = pl.ds(v * LANES, LANES)
                    eidx[p, 0, b, sl] = eidx[p, 0, b, sl] + coff

        def gather_start(p, b):
            pltpu.async_copy(xs_hbm.at[eidx.at[p, 0, b]], rows.at[b],
                             gsem[b])

        def gather_wait(p, b):
            pltpu.make_async_copy(xs_hbm.at[eidx.at[p, 0, b]], rows.at[b],
                                  gsem[b]).wait()

        def scatter_start(p, b):
            pltpu.async_copy(rows.at[b], acc.at[eidx.at[p, 1, b]], ssem[b],
                             add=True)

        def scatter_wait(p, b):
            pltpu.make_async_copy(rows.at[b], acc.at[eidx.at[p, 1, b]],
                                  ssem[b]).wait()

        idx_load(0, 0)
        for b in range(NBUF):
            gather_start(0, b)

        @pl.loop(0, n_rounds, step=2)
        def _(r0):
            for p in range(2):
                idx_load(r0 + p + 1, 1 - p)
                for b in range(NBUF):
                    gather_wait(p, b)
                    scatter_start(p, b)
                for b in range(NBUF):
                    scatter_wait(p, b)
                    gather_start(1 - p, b)

        for b in range(NBUF):
            gather_wait(0, b)

        plsc.subcore_barrier()
        pltpu.sync_copy(acc.at[pl.ds(s * nrt, nrt)],
                        out_hbm.at[c, pl.ds(s * nrt, nrt)])

    return agg_kernel(xs_flat, edges3)


def _prep_body(x_ref, w_ref, degp_ref, xs_ref, dis_ref):
    degp = degp_ref[...]
    deg = degp[0, :, 0] + degp[1, :, 0] + 1.0
    dis = lax.rsqrt(deg)
    xw = jnp.dot(x_ref[...], w_ref[...], preferred_element_type=jnp.float32)
    xs = xw * dis[:, None]
    half = xs.shape[1] // 2
    xs_ref[0] = xs[:, :half]
    xs_ref[1] = xs[:, half:]
    dis_ref[...] = dis[:, None]


def _layer_mm_body(n_rows, acc_ref, dis_ref, b_ref, g_ref, be_ref, w_ref,
                   out_ref, ssum):
    ph = pl.program_id(0)
    i = pl.program_id(1)
    dis = dis_ref[...][:, 0]
    z = acc_ref[...] * dis[None, :, None] + b_ref[...][:, None, :]

    @pl.when(jnp.logical_and(ph == 0, i == 0))
    def _():
        ssum[...] = jnp.zeros_like(ssum)

    @pl.when(ph == 0)
    def _():
        ssum[0] += jnp.sum(z, axis=1)
        ssum[1] += jnp.sum(z * z, axis=1)

    @pl.when(ph == 1)
    def _():
        st = ssum[...]
        mean = st[0] / n_rows
        var = st[1] / n_rows - mean * mean
        scale = lax.rsqrt(var + EPS) * g_ref[...]
        shift = be_ref[...] - mean * scale
        h = jnp.maximum(z * scale[:, None, :] + shift[:, None, :], 0.0)
        hf = jnp.concatenate([h[0], h[1]], axis=1)
        xw = jnp.dot(hf, w_ref[...], preferred_element_type=jnp.float32)
        xs = xw * dis[:, None]
        half = xw.shape[1] // 2
        out_ref[0] = xs[:, :half]
        out_ref[1] = xs[:, half:]


def _layer_body(n_rows, acc_ref, dis_ref, b_ref, g_ref, be_ref, out_ref,
                ssum):
    ph = pl.program_id(0)
    i = pl.program_id(1)
    dis = dis_ref[...][:, 0]
    z = acc_ref[...] * dis[None, :, None] + b_ref[...][:, None, :]

    @pl.when(jnp.logical_and(ph == 0, i == 0))
    def _():
        ssum[...] = jnp.zeros_like(ssum)

    @pl.when(ph == 0)
    def _():
        ssum[0] += jnp.sum(z, axis=1)
        ssum[1] += jnp.sum(z * z, axis=1)

    @pl.when(ph == 1)
    def _():
        st = ssum[...]
        mean = st[0] / n_rows
        var = st[1] / n_rows - mean * mean
        scale = lax.rsqrt(var + EPS) * g_ref[...]
        shift = be_ref[...] - mean * scale
        h = jnp.maximum(z * scale[:, None, :] + shift[:, None, :], 0.0)
        out_ref[...] = h * dis[None, :, None]


def _final_body(acc_ref, dis_ref, wmu_ref, bmu_ref, wls_ref, bls_ref,
                mu_ref, ls_ref):
    dis = dis_ref[...]
    a = acc_ref[...]
    t = jnp.concatenate([a[0], a[1]], axis=1) * dis
    mu_ref[...] = (jnp.dot(t, wmu_ref[...], preferred_element_type=jnp.float32)
                   + bmu_ref[...])
    ls_ref[...] = (jnp.dot(t, wls_ref[...], preferred_element_type=jnp.float32)
                   + bls_ref[...])


def _full_spec(shape, ndim_grid=1):
    zeros = (0,) * len(shape)
    if ndim_grid == 2:
        return pl.BlockSpec(shape, lambda p, i: zeros)
    return pl.BlockSpec(shape, lambda i: zeros)


def _row_spec(nd_shape, ndim_grid=1):
    if len(nd_shape) == 3:
        if ndim_grid == 2:
            return pl.BlockSpec((nd_shape[0], ROWS, nd_shape[2]),
                                lambda p, i: (0, i, 0))
        return pl.BlockSpec((nd_shape[0], ROWS, nd_shape[2]),
                            lambda i: (0, i, 0))
    if ndim_grid == 2:
        return pl.BlockSpec((ROWS, nd_shape[1]), lambda p, i: (i, 0))
    return pl.BlockSpec((ROWS, nd_shape[1]), lambda i: (i, 0))


def kernel(x, edge_index, W1, b1, g1, be1, W2, b2, g2, be2, Wmu, bmu, Wls,
           bls):
    n, in_ch = x.shape
    hid = W1.shape[1]
    lat = Wmu.shape[1]
    half = hid // 2
    e = edge_index.shape[1]
    idt = edge_index.dtype

    n_blocks = n // ROWS
    nrt = -(-(n + 1) // (NS * 8)) * 8
    n_acc = nrt * NS

    e_unit = NC * NS * CHUNK * NBUF * 2
    e_work = -(-e // e_unit) * e_unit
    pad = e_work + NBUF * CHUNK - e
    pad_col = jnp.concatenate(
        [jnp.zeros((1, pad), idt), jnp.full((1, pad), n, idt)])
    edges3 = jnp.concatenate([edge_index, pad_col], axis=1).reshape(
        2, -1, CHUNK)

    degp = _sc_degree(edges3, jnp.zeros((n_acc, 8), jnp.float32),
                      jnp.ones((CHUNK, 8), jnp.float32), n_acc, e_work)

    xs1, dis = pl.pallas_call(
        _prep_body,
        grid=(n_blocks,),
        in_specs=[_row_spec((n, in_ch)), _full_spec(W1.shape),
                  _row_spec((NC, n_acc, 8))],
        out_specs=[_row_spec((NC, n_acc, half)), _row_spec((n_acc, 1))],
        out_shape=[jax.ShapeDtypeStruct((NC, n_acc, half), jnp.float32),
                   jax.ShapeDtypeStruct((n_acc, 1), jnp.float32)],
    )(x, W1, degp)

    nf = float(n)

    acc1 = _sc_aggregate(xs1.reshape(NC * n_acc, half), edges3, n_acc, half,
                         e_work)
    xs2 = pl.pallas_call(
        functools.partial(_layer_mm_body, nf),
        grid=(2, n_blocks),
        in_specs=[_row_spec((NC, n_acc, half), 2), _row_spec((n_acc, 1), 2),
                  _full_spec((NC, half), 2), _full_spec((NC, half), 2),
                  _full_spec((NC, half), 2), _full_spec(W2.shape, 2)],
        out_specs=_row_spec((NC, n_acc, half), 2),
        out_shape=jax.ShapeDtypeStruct((NC, n_acc, half), jnp.float32),
        scratch_shapes=[pltpu.VMEM((2, NC, half), jnp.float32)],
    )(acc1, dis, b1.reshape(NC, half), g1.reshape(NC, half),
      be1.reshape(NC, half), W2)

    acc2 = _sc_aggregate(xs2.reshape(NC * n_acc, half), edges3, n_acc, half,
                         e_work)
    xs3 = pl.pallas_call(
        functools.partial(_layer_body, nf),
        grid=(2, n_blocks),
        in_specs=[_row_spec((NC, n_acc, half), 2), _row_spec((n_acc, 1), 2),
                  _full_spec((NC, half), 2), _full_spec((NC, half), 2),
                  _full_spec((NC, half), 2)],
        out_specs=_row_spec((NC, n_acc, half), 2),
        out_shape=jax.ShapeDtypeStruct((NC, n_acc, half), jnp.float32),
        scratch_shapes=[pltpu.VMEM((2, NC, half), jnp.float32)],
    )(acc2, dis, b2.reshape(NC, half), g2.reshape(NC, half),
      be2.reshape(NC, half))

    acc3 = _sc_aggregate(xs3.reshape(NC * n_acc, half), edges3, n_acc, half,
                         e_work)
    mu, ls = pl.pallas_call(
        _final_body,
        grid=(n_blocks,),
        in_specs=[_row_spec((NC, n_acc, half)), _row_spec((n_acc, 1)),
                  _full_spec(Wmu.shape), _full_spec((1, lat)),
                  _full_spec(Wls.shape), _full_spec((1, lat))],
        out_specs=[_row_spec((n, lat)), _row_spec((n, lat))],
        out_shape=[jax.ShapeDtypeStruct((n, lat), jnp.float32),
                   jax.ShapeDtypeStruct((n, lat), jnp.float32)],
    )(acc3, dis, Wmu, bmu.reshape(1, lat), Wls, bls.reshape(1, lat))

    return (mu, ls)


# speedup vs baseline: 1.3611x; 1.3611x over previous
import functools

import jax
import jax.numpy as jnp
from jax import lax
from jax.experimental import pallas as pl
from jax.experimental.pallas import tpu as pltpu
from jax.experimental.pallas import tpu_sc as plsc

NC = 2
NS = 16
LANES = 16
CHUNK = 128
NBUF = 4
ROWS = 1000
EPS = 1e-5


def _sc_degree(edges3, zeros8, ones8, n_acc, e_work):
    e_per_tile = e_work // (NC * NS)
    n_rounds = e_per_tile // (CHUNK * NBUF)
    cpt = e_per_tile // CHUNK
    nrt = n_acc // NS
    mesh = plsc.VectorSubcoreMesh(core_axis_name="c", subcore_axis_name="s")

    @functools.partial(
        pl.kernel,
        out_type=jax.ShapeDtypeStruct((NC, n_acc, 8), jnp.float32),
        mesh=mesh,
        compiler_params=pltpu.CompilerParams(use_tc_tiling_on_sc=False),
        scratch_types=[
            pltpu.VMEM((2, NBUF, CHUNK), jnp.int32),
            pltpu.VMEM((CHUNK, 8), jnp.float32),
            pltpu.VMEM_SHARED((n_acc, 8), jnp.float32),
        ] + [pltpu.SemaphoreType.DMA] * NBUF,
    )
    def deg_kernel(edges_hbm, zeros_hbm, ones_hbm, out_hbm, didx, ones_v,
                   acc, *ssem):
        c = lax.axis_index("c")
        s = lax.axis_index("s")
        pltpu.sync_copy(zeros_hbm.at[pl.ds(s * nrt, nrt)],
                        acc.at[pl.ds(s * nrt, nrt)])
        pltpu.sync_copy(ones_hbm, ones_v)
        plsc.subcore_barrier()
        cb = (c * NS + s) * cpt

        def idx_load(r, p):
            pltpu.sync_copy(edges_hbm.at[1, pl.ds(cb + r * NBUF, NBUF)],
                            didx.at[p])

        def scatter_start(p, b):
            pltpu.async_copy(ones_v, acc.at[didx.at[p, b]], ssem[b],
                             add=True)

        def scatter_wait(p, b):
            pltpu.make_async_copy(ones_v, acc.at[didx.at[p, b]],
                                  ssem[b]).wait()

        def round_body(r, p):
            for b in range(NBUF):
                scatter_start(p, b)
            idx_load(r + 1, 1 - p)
            for b in range(NBUF):
                scatter_wait(p, b)

        idx_load(0, 0)
        n_even = n_rounds - (n_rounds % 2)

        @pl.loop(0, n_even, step=2)
        def _(r0):
            round_body(r0, 0)
            round_body(r0 + 1, 1)

        if n_rounds % 2:
            round_body(n_rounds - 1, 0)

        plsc.subcore_barrier()
        pltpu.sync_copy(acc.at[pl.ds(s * nrt, nrt)],
                        out_hbm.at[c, pl.ds(s * nrt, nrt)])

    return deg_kernel(edges3, zeros8, ones8)


def _sc_aggregate(xs_flat, edges3, n_acc, half, e_work):
    e_per_tile = e_work // NS
    n_rounds = e_per_tile // (CHUNK * NBUF)
    cpt = e_per_tile // CHUNK
    nrt = n_acc // NS
    mesh = plsc.VectorSubcoreMesh(core_axis_name="c", subcore_axis_name="s")

    @functools.partial(
        pl.kernel,
        out_type=jax.ShapeDtypeStruct((NC, n_acc, half), jnp.float32),
        mesh=mesh,
        compiler_params=pltpu.CompilerParams(use_tc_tiling_on_sc=False),
        scratch_types=[
            pltpu.VMEM((2, 2, NBUF, CHUNK), jnp.int32),
            pltpu.VMEM((NBUF, CHUNK, half), jnp.float32),
            pltpu.VMEM_SHARED((n_acc, half), jnp.float32),
        ] + [pltpu.SemaphoreType.DMA] * (2 * NBUF),
    )
    def agg_kernel(xs_hbm, edges_hbm, out_hbm, eidx, rows, acc, *sems):
        gsem, ssem = sems[:NBUF], sems[NBUF:]
        c = lax.axis_index("c")
        s = lax.axis_index("s")
        coff = c * n_acc
        pltpu.sync_copy(xs_hbm.at[pl.ds(coff + s * nrt, nrt)],
                        acc.at[pl.ds(s * nrt, nrt)])
        plsc.subcore_barrier()
        cb = s * cpt

        def idx_load(r, p):
            pltpu.sync_copy(edges_hbm.at[:, pl.ds(cb + r * NBUF, NBUF)],
                            eidx.at[p])
            for b in range(NBUF):
                for v in range(CHUNK // LANES):
                    sl = pl.ds(v * LANES, LANES)
                    eidx[p, 0, b, sl] = eidx[p, 0, b, sl] + coff

        def gather_start(p, b):
            pltpu.async_copy(xs_hbm.at[eidx.at[p, 0, b]], rows.at[b],
                             gsem[b])

        def gather_wait(p, b):
            pltpu.make_async_copy(xs_hbm.at[eidx.at[p, 0, b]], rows.at[b],
                                  gsem[b]).wait()

        def scatter_start(p, b):
            pltpu.async_copy(rows.at[b], acc.at[eidx.at[p, 1, b]], ssem[b],
                             add=True)

        def scatter_wait(p, b):
            pltpu.make_async_copy(rows.at[b], acc.at[eidx.at[p, 1, b]],
                                  ssem[b]).wait()

        def round_body(r, p):
            for b in range(NBUF):
                gather_wait(p, b)
                scatter_start(p, b)
            idx_load(r + 1, 1 - p)
            for b in range(NBUF):
                scatter_wait(p, b)
                gather_start(1 - p, b)

        idx_load(0, 0)
        for b in range(NBUF):
            gather_start(0, b)
        n_even = n_rounds - (n_rounds % 2)

        @pl.loop(0, n_even, step=2)
        def _(r0):
            round_body(r0, 0)
            round_body(r0 + 1, 1)

        if n_rounds % 2:
            round_body(n_rounds - 1, 0)

        for b in range(NBUF):
            gather_wait(n_rounds % 2, b)

        plsc.subcore_barrier()
        pltpu.sync_copy(acc.at[pl.ds(s * nrt, nrt)],
                        out_hbm.at[c, pl.ds(s * nrt, nrt)])

    return agg_kernel(xs_flat, edges3)


def _prep_body(x_ref, w_ref, degp_ref, xs_ref, dis_ref):
    degp = degp_ref[...]
    deg = degp[0, :, 0] + degp[1, :, 0] + 1.0
    dis = lax.rsqrt(deg)
    xw = jnp.dot(x_ref[...], w_ref[...], preferred_element_type=jnp.float32)
    xs = xw * dis[:, None]
    half = xs.shape[1] // 2
    xs_ref[0] = xs[:, :half]
    xs_ref[1] = xs[:, half:]
    dis_ref[...] = dis[:, None]


def _layer_mm_body(n_rows, acc_ref, dis_ref, b_ref, g_ref, be_ref, w_ref,
                   out_ref, ssum):
    ph = pl.program_id(0)
    i = pl.program_id(1)
    dis = dis_ref[...][:, 0]
    z = acc_ref[...] * dis[None, :, None] + b_ref[...][:, None, :]

    @pl.when(jnp.logical_and(ph == 0, i == 0))
    def _():
        ssum[...] = jnp.zeros_like(ssum)

    @pl.when(ph == 0)
    def _():
        ssum[0] += jnp.sum(z, axis=1)
        ssum[1] += jnp.sum(z * z, axis=1)

    @pl.when(ph == 1)
    def _():
        st = ssum[...]
        mean = st[0] / n_rows
        var = st[1] / n_rows - mean * mean
        scale = lax.rsqrt(var + EPS) * g_ref[...]
        shift = be_ref[...] - mean * scale
        h = jnp.maximum(z * scale[:, None, :] + shift[:, None, :], 0.0)
        hf = jnp.concatenate([h[0], h[1]], axis=1)
        xw = jnp.dot(hf, w_ref[...], preferred_element_type=jnp.float32)
        xs = xw * dis[:, None]
        half = xw.shape[1] // 2
        out_ref[0] = xs[:, :half]
        out_ref[1] = xs[:, half:]


def _layer_body(n_rows, acc_ref, dis_ref, b_ref, g_ref, be_ref, out_ref,
                ssum):
    ph = pl.program_id(0)
    i = pl.program_id(1)
    dis = dis_ref[...][:, 0]
    z = acc_ref[...] * dis[None, :, None] + b_ref[...][:, None, :]

    @pl.when(jnp.logical_and(ph == 0, i == 0))
    def _():
        ssum[...] = jnp.zeros_like(ssum)

    @pl.when(ph == 0)
    def _():
        ssum[0] += jnp.sum(z, axis=1)
        ssum[1] += jnp.sum(z * z, axis=1)

    @pl.when(ph == 1)
    def _():
        st = ssum[...]
        mean = st[0] / n_rows
        var = st[1] / n_rows - mean * mean
        scale = lax.rsqrt(var + EPS) * g_ref[...]
        shift = be_ref[...] - mean * scale
        h = jnp.maximum(z * scale[:, None, :] + shift[:, None, :], 0.0)
        out_ref[...] = h * dis[None, :, None]


def _final_body(acc_ref, dis_ref, wmu_ref, bmu_ref, wls_ref, bls_ref,
                mu_ref, ls_ref):
    dis = dis_ref[...]
    a = acc_ref[...]
    t = jnp.concatenate([a[0], a[1]], axis=1) * dis
    mu_ref[...] = (jnp.dot(t, wmu_ref[...], preferred_element_type=jnp.float32)
                   + bmu_ref[...])
    ls_ref[...] = (jnp.dot(t, wls_ref[...], preferred_element_type=jnp.float32)
                   + bls_ref[...])


def _full_spec(shape, ndim_grid=1):
    zeros = (0,) * len(shape)
    if ndim_grid == 2:
        return pl.BlockSpec(shape, lambda p, i: zeros)
    return pl.BlockSpec(shape, lambda i: zeros)


def _row_spec(nd_shape, ndim_grid=1):
    if len(nd_shape) == 3:
        if ndim_grid == 2:
            return pl.BlockSpec((nd_shape[0], ROWS, nd_shape[2]),
                                lambda p, i: (0, i, 0))
        return pl.BlockSpec((nd_shape[0], ROWS, nd_shape[2]),
                            lambda i: (0, i, 0))
    if ndim_grid == 2:
        return pl.BlockSpec((ROWS, nd_shape[1]), lambda p, i: (i, 0))
    return pl.BlockSpec((ROWS, nd_shape[1]), lambda i: (i, 0))


def kernel(x, edge_index, W1, b1, g1, be1, W2, b2, g2, be2, Wmu, bmu, Wls,
           bls):
    n, in_ch = x.shape
    hid = W1.shape[1]
    lat = Wmu.shape[1]
    half = hid // 2
    e = edge_index.shape[1]
    idt = edge_index.dtype

    n_blocks = n // ROWS
    nrt = -(-(n + 1) // (NS * 8)) * 8
    n_acc = nrt * NS

    e_unit = NC * NS * CHUNK
    e_work = -(-e // e_unit) * e_unit
    pad = e_work + NBUF * CHUNK - e
    trash = n + jnp.arange(pad, dtype=idt) % (n_acc - n)
    pad_col = jnp.concatenate(
        [jnp.zeros((1, pad), idt), trash[None, :]])
    edges3 = jnp.concatenate([edge_index, pad_col], axis=1).reshape(
        2, -1, CHUNK)

    degp = _sc_degree(edges3, jnp.zeros((n_acc, 8), jnp.float32),
                      jnp.ones((CHUNK, 8), jnp.float32), n_acc, e_work)

    xs1, dis = pl.pallas_call(
        _prep_body,
        grid=(n_blocks,),
        in_specs=[_row_spec((n, in_ch)), _full_spec(W1.shape),
                  _row_spec((NC, n_acc, 8))],
        out_specs=[_row_spec((NC, n_acc, half)), _row_spec((n_acc, 1))],
        out_shape=[jax.ShapeDtypeStruct((NC, n_acc, half), jnp.float32),
                   jax.ShapeDtypeStruct((n_acc, 1), jnp.float32)],
    )(x, W1, degp)

    nf = float(n)

    acc1 = _sc_aggregate(xs1.reshape(NC * n_acc, half), edges3, n_acc, half,
                         e_work)
    xs2 = pl.pallas_call(
        functools.partial(_layer_mm_body, nf),
        grid=(2, n_blocks),
        in_specs=[_row_spec((NC, n_acc, half), 2), _row_spec((n_acc, 1), 2),
                  _full_spec((NC, half), 2), _full_spec((NC, half), 2),
                  _full_spec((NC, half), 2), _full_spec(W2.shape, 2)],
        out_specs=_row_spec((NC, n_acc, half), 2),
        out_shape=jax.ShapeDtypeStruct((NC, n_acc, half), jnp.float32),
        scratch_shapes=[pltpu.VMEM((2, NC, half), jnp.float32)],
    )(acc1, dis, b1.reshape(NC, half), g1.reshape(NC, half),
      be1.reshape(NC, half), W2)

    acc2 = _sc_aggregate(xs2.reshape(NC * n_acc, half), edges3, n_acc, half,
                         e_work)
    xs3 = pl.pallas_call(
        functools.partial(_layer_body, nf),
        grid=(2, n_blocks),
        in_specs=[_row_spec((NC, n_acc, half), 2), _row_spec((n_acc, 1), 2),
                  _full_spec((NC, half), 2), _full_spec((NC, half), 2),
                  _full_spec((NC, half), 2)],
        out_specs=_row_spec((NC, n_acc, half), 2),
        out_shape=jax.ShapeDtypeStruct((NC, n_acc, half), jnp.float32),
        scratch_shapes=[pltpu.VMEM((2, NC, half), jnp.float32)],
    )(acc2, dis, b2.reshape(NC, half), g2.reshape(NC, half),
      be2.reshape(NC, half))

    acc3 = _sc_aggregate(xs3.reshape(NC * n_acc, half), edges3, n_acc, half,
                         e_work)
    mu, ls = pl.pallas_call(
        _final_body,
        grid=(n_blocks,),
        in_specs=[_row_spec((NC, n_acc, half)), _row_spec((n_acc, 1)),
                  _full_spec(Wmu.shape), _full_spec((1, lat)),
                  _full_spec(Wls.shape), _full_spec((1, lat))],
        out_specs=[_row_spec((n, lat)), _row_spec((n, lat))],
        out_shape=[jax.ShapeDtypeStruct((n, lat), jnp.float32),
                   jax.ShapeDtypeStruct((n, lat), jnp.float32)],
    )(acc3, dis, Wmu, bmu.reshape(1, lat), Wls, bls.reshape(1, lat))

    return (mu, ls)


# speedup vs baseline: 1.4545x; 1.0686x over previous
import functools

import jax
import jax.numpy as jnp
from jax import lax
from jax.experimental import pallas as pl
from jax.experimental.pallas import tpu as pltpu
from jax.experimental.pallas import tpu_sc as plsc

NC = 2
NS = 16
LANES = 16
CHUNK = 128
NBUF = 4
ROWS = 2000
EPS = 1e-5


def _sc_degree(edges3, zeros8, ones8, n_acc, e_work):
    e_per_tile = e_work // (NC * NS)
    n_rounds = e_per_tile // (CHUNK * NBUF)
    cpt = e_per_tile // CHUNK
    nrt = n_acc // NS
    mesh = plsc.VectorSubcoreMesh(core_axis_name="c", subcore_axis_name="s")

    @functools.partial(
        pl.kernel,
        out_type=jax.ShapeDtypeStruct((NC, n_acc, 8), jnp.float32),
        mesh=mesh,
        compiler_params=pltpu.CompilerParams(use_tc_tiling_on_sc=False),
        scratch_types=[
            pltpu.VMEM((2, NBUF, CHUNK), jnp.int32),
            pltpu.VMEM((CHUNK, 8), jnp.float32),
            pltpu.VMEM_SHARED((n_acc, 8), jnp.float32),
        ] + [pltpu.SemaphoreType.DMA] * NBUF,
    )
    def deg_kernel(edges_hbm, zeros_hbm, ones_hbm, out_hbm, didx, ones_v,
                   acc, *ssem):
        c = lax.axis_index("c")
        s = lax.axis_index("s")
        pltpu.sync_copy(zeros_hbm.at[pl.ds(s * nrt, nrt)],
                        acc.at[pl.ds(s * nrt, nrt)])
        pltpu.sync_copy(ones_hbm, ones_v)
        plsc.subcore_barrier()
        cb = (c * NS + s) * cpt

        def idx_load(r, p):
            pltpu.sync_copy(edges_hbm.at[1, pl.ds(cb + r * NBUF, NBUF)],
                            didx.at[p])

        def scatter_start(p, b):
            pltpu.async_copy(ones_v, acc.at[didx.at[p, b]], ssem[b],
                             add=True)

        def scatter_wait(p, b):
            pltpu.make_async_copy(ones_v, acc.at[didx.at[p, b]],
                                  ssem[b]).wait()

        def round_body(r, p):
            for b in range(NBUF):
                scatter_start(p, b)
            idx_load(r + 1, 1 - p)
            for b in range(NBUF):
                scatter_wait(p, b)

        idx_load(0, 0)
        n_even = n_rounds - (n_rounds % 2)

        @pl.loop(0, n_even, step=2)
        def _(r0):
            round_body(r0, 0)
            round_body(r0 + 1, 1)

        if n_rounds % 2:
            round_body(n_rounds - 1, 0)

        plsc.subcore_barrier()
        pltpu.sync_copy(acc.at[pl.ds(s * nrt, nrt)],
                        out_hbm.at[c, pl.ds(s * nrt, nrt)])

    return deg_kernel(edges3, zeros8, ones8)


def _sc_aggregate(xs_flat, edges3, n_acc, half, e_work):
    e_per_tile = e_work // NS
    n_rounds = e_per_tile // (CHUNK * NBUF)
    cpt = e_per_tile // CHUNK
    nrt = n_acc // NS
    mesh = plsc.VectorSubcoreMesh(core_axis_name="c", subcore_axis_name="s")

    @functools.partial(
        pl.kernel,
        out_type=jax.ShapeDtypeStruct((NC, n_acc, half), jnp.float32),
        mesh=mesh,
        compiler_params=pltpu.CompilerParams(use_tc_tiling_on_sc=False),
        scratch_types=[
            pltpu.VMEM((2, 2, NBUF, CHUNK), jnp.int32),
            pltpu.VMEM((NBUF, CHUNK, half), jnp.float32),
            pltpu.VMEM_SHARED((n_acc, half), jnp.float32),
        ] + [pltpu.SemaphoreType.DMA] * (2 * NBUF),
    )
    def agg_kernel(xs_hbm, edges_hbm, out_hbm, eidx, rows, acc, *sems):
        gsem, ssem = sems[:NBUF], sems[NBUF:]
        c = lax.axis_index("c")
        s = lax.axis_index("s")
        coff = c * n_acc
        pltpu.sync_copy(xs_hbm.at[pl.ds(coff + s * nrt, nrt)],
                        acc.at[pl.ds(s * nrt, nrt)])
        plsc.subcore_barrier()
        cb = s * cpt

        def idx_load(r, p):
            pltpu.sync_copy(edges_hbm.at[:, pl.ds(cb + r * NBUF, NBUF)],
                            eidx.at[p])
            for b in range(NBUF):
                for v in range(CHUNK // LANES):
                    sl = pl.ds(v * LANES, LANES)
                    eidx[p, 0, b, sl] = eidx[p, 0, b, sl] + coff

        def gather_start(p, b):
            pltpu.async_copy(xs_hbm.at[eidx.at[p, 0, b]], rows.at[b],
                             gsem[b])

        def gather_wait(p, b):
            pltpu.make_async_copy(xs_hbm.at[eidx.at[p, 0, b]], rows.at[b],
                                  gsem[b]).wait()

        def scatter_start(p, b):
            pltpu.async_copy(rows.at[b], acc.at[eidx.at[p, 1, b]], ssem[b],
                             add=True)

        def scatter_wait(p, b):
            pltpu.make_async_copy(rows.at[b], acc.at[eidx.at[p, 1, b]],
                                  ssem[b]).wait()

        def round_body(r, p):
            for b in range(NBUF):
                gather_wait(p, b)
                scatter_start(p, b)
            idx_load(r + 1, 1 - p)
            for b in range(NBUF):
                scatter_wait(p, b)
                gather_start(1 - p, b)

        idx_load(0, 0)
        for b in range(NBUF):
            gather_start(0, b)
        n_even = n_rounds - (n_rounds % 2)

        @pl.loop(0, n_even, step=2)
        def _(r0):
            round_body(r0, 0)
            round_body(r0 + 1, 1)

        if n_rounds % 2:
            round_body(n_rounds - 1, 0)

        for b in range(NBUF):
            gather_wait(n_rounds % 2, b)

        plsc.subcore_barrier()
        pltpu.sync_copy(acc.at[pl.ds(s * nrt, nrt)],
                        out_hbm.at[c, pl.ds(s * nrt, nrt)])

    return agg_kernel(xs_flat, edges3)


def _prep_body(x_ref, w_ref, degp_ref, xs_ref, dis_ref):
    degp = degp_ref[...]
    deg = degp[0, :, 0] + degp[1, :, 0] + 1.0
    dis = lax.rsqrt(deg)
    xw = jnp.dot(x_ref[...], w_ref[...], preferred_element_type=jnp.float32)
    xs = xw * dis[:, None]
    half = xs.shape[1] // 2
    xs_ref[0] = xs[:, :half]
    xs_ref[1] = xs[:, half:]
    dis_ref[...] = dis[:, None]


def _layer_mm_body(n_rows, acc_ref, dis_ref, b_ref, g_ref, be_ref, w_ref,
                   out_ref, ssum):
    ph = pl.program_id(0)
    i = pl.program_id(1)
    dis = dis_ref[...][:, 0]
    z = acc_ref[...] * dis[None, :, None] + b_ref[...][:, None, :]

    @pl.when(jnp.logical_and(ph == 0, i == 0))
    def _():
        ssum[...] = jnp.zeros_like(ssum)

    @pl.when(ph == 0)
    def _():
        ssum[0] += jnp.sum(z, axis=1)
        ssum[1] += jnp.sum(z * z, axis=1)

    @pl.when(ph == 1)
    def _():
        st = ssum[...]
        mean = st[0] / n_rows
        var = st[1] / n_rows - mean * mean
        scale = lax.rsqrt(var + EPS) * g_ref[...]
        shift = be_ref[...] - mean * scale
        h = jnp.maximum(z * scale[:, None, :] + shift[:, None, :], 0.0)
        hf = jnp.concatenate([h[0], h[1]], axis=1)
        xw = jnp.dot(hf, w_ref[...], preferred_element_type=jnp.float32)
        xs = xw * dis[:, None]
        half = xw.shape[1] // 2
        out_ref[0] = xs[:, :half]
        out_ref[1] = xs[:, half:]


def _layer_body(n_rows, acc_ref, dis_ref, b_ref, g_ref, be_ref, out_ref,
                ssum):
    ph = pl.program_id(0)
    i = pl.program_id(1)
    dis = dis_ref[...][:, 0]
    z = acc_ref[...] * dis[None, :, None] + b_ref[...][:, None, :]

    @pl.when(jnp.logical_and(ph == 0, i == 0))
    def _():
        ssum[...] = jnp.zeros_like(ssum)

    @pl.when(ph == 0)
    def _():
        ssum[0] += jnp.sum(z, axis=1)
        ssum[1] += jnp.sum(z * z, axis=1)

    @pl.when(ph == 1)
    def _():
        st = ssum[...]
        mean = st[0] / n_rows
        var = st[1] / n_rows - mean * mean
        scale = lax.rsqrt(var + EPS) * g_ref[...]
        shift = be_ref[...] - mean * scale
        h = jnp.maximum(z * scale[:, None, :] + shift[:, None, :], 0.0)
        out_ref[...] = h * dis[None, :, None]


def _final_body(acc_ref, dis_ref, wmu_ref, bmu_ref, wls_ref, bls_ref,
                mu_ref, ls_ref):
    dis = dis_ref[...]
    a = acc_ref[...]
    t = jnp.concatenate([a[0], a[1]], axis=1) * dis
    mu_ref[...] = (jnp.dot(t, wmu_ref[...], preferred_element_type=jnp.float32)
                   + bmu_ref[...])
    ls_ref[...] = (jnp.dot(t, wls_ref[...], preferred_element_type=jnp.float32)
                   + bls_ref[...])


def _full_spec(shape, ndim_grid=1):
    zeros = (0,) * len(shape)
    if ndim_grid == 2:
        return pl.BlockSpec(shape, lambda p, i: zeros)
    return pl.BlockSpec(shape, lambda i: zeros)


def _row_spec(nd_shape, ndim_grid=1):
    if len(nd_shape) == 3:
        if ndim_grid == 2:
            return pl.BlockSpec((nd_shape[0], ROWS, nd_shape[2]),
                                lambda p, i: (0, i, 0))
        return pl.BlockSpec((nd_shape[0], ROWS, nd_shape[2]),
                            lambda i: (0, i, 0))
    if ndim_grid == 2:
        return pl.BlockSpec((ROWS, nd_shape[1]), lambda p, i: (i, 0))
    return pl.BlockSpec((ROWS, nd_shape[1]), lambda i: (i, 0))


def kernel(x, edge_index, W1, b1, g1, be1, W2, b2, g2, be2, Wmu, bmu, Wls,
           bls):
    n, in_ch = x.shape
    hid = W1.shape[1]
    lat = Wmu.shape[1]
    half = hid // 2
    e = edge_index.shape[1]
    idt = edge_index.dtype

    n_blocks = n // ROWS
    nrt = -(-(n + 1) // (NS * 8)) * 8
    n_acc = nrt * NS

    e_unit = NC * NS * CHUNK
    e_work = -(-e // e_unit) * e_unit
    pad = e_work + NBUF * CHUNK - e
    trash = n + jnp.arange(pad, dtype=idt) % (n_acc - n)
    pad_col = jnp.concatenate(
        [jnp.zeros((1, pad), idt), trash[None, :]])
    edges3 = jnp.concatenate([edge_index, pad_col], axis=1).reshape(
        2, -1, CHUNK)

    degp = _sc_degree(edges3, jnp.zeros((n_acc, 8), jnp.float32),
                      jnp.ones((CHUNK, 8), jnp.float32), n_acc, e_work)

    xs1, dis = pl.pallas_call(
        _prep_body,
        grid=(n_blocks,),
        in_specs=[_row_spec((n, in_ch)), _full_spec(W1.shape),
                  _row_spec((NC, n_acc, 8))],
        out_specs=[_row_spec((NC, n_acc, half)), _row_spec((n_acc, 1))],
        out_shape=[jax.ShapeDtypeStruct((NC, n_acc, half), jnp.float32),
                   jax.ShapeDtypeStruct((n_acc, 1), jnp.float32)],
    )(x, W1, degp)

    nf = float(n)

    acc1 = _sc_aggregate(xs1.reshape(NC * n_acc, half), edges3, n_acc, half,
                         e_work)
    xs2 = pl.pallas_call(
        functools.partial(_layer_mm_body, nf),
        grid=(2, n_blocks),
        in_specs=[_row_spec((NC, n_acc, half), 2), _row_spec((n_acc, 1), 2),
                  _full_spec((NC, half), 2), _full_spec((NC, half), 2),
                  _full_spec((NC, half), 2), _full_spec(W2.shape, 2)],
        out_specs=_row_spec((NC, n_acc, half), 2),
        out_shape=jax.ShapeDtypeStruct((NC, n_acc, half), jnp.float32),
        scratch_shapes=[pltpu.VMEM((2, NC, half), jnp.float32)],
    )(acc1, dis, b1.reshape(NC, half), g1.reshape(NC, half),
      be1.reshape(NC, half), W2)

    acc2 = _sc_aggregate(xs2.reshape(NC * n_acc, half), edges3, n_acc, half,
                         e_work)
    xs3 = pl.pallas_call(
        functools.partial(_layer_body, nf),
        grid=(2, n_blocks),
        in_specs=[_row_spec((NC, n_acc, half), 2), _row_spec((n_acc, 1), 2),
                  _full_spec((NC, half), 2), _full_spec((NC, half), 2),
                  _full_spec((NC, half), 2)],
        out_specs=_row_spec((NC, n_acc, half), 2),
        out_shape=jax.ShapeDtypeStruct((NC, n_acc, half), jnp.float32),
        scratch_shapes=[pltpu.VMEM((2, NC, half), jnp.float32)],
    )(acc2, dis, b2.reshape(NC, half), g2.reshape(NC, half),
      be2.reshape(NC, half))

    acc3 = _sc_aggregate(xs3.reshape(NC * n_acc, half), edges3, n_acc, half,
                         e_work)
    mu, ls = pl.pallas_call(
        _final_body,
        grid=(n_blocks,),
        in_specs=[_row_spec((NC, n_acc, half)), _row_spec((n_acc, 1)),
                  _full_spec(Wmu.shape), _full_spec((1, lat)),
                  _full_spec(Wls.shape), _full_spec((1, lat))],
        out_specs=[_row_spec((n, lat)), _row_spec((n, lat))],
        out_shape=[jax.ShapeDtypeStruct((n, lat), jnp.float32),
                   jax.ShapeDtypeStruct((n, lat), jnp.float32)],
    )(acc3, dis, Wmu, bmu.reshape(1, lat), Wls, bls.reshape(1, lat))

    return (mu, ls)


# speedup vs baseline: 1.4822x; 1.0191x over previous
import functools

import jax
import jax.numpy as jnp
from jax import lax
from jax.experimental import pallas as pl
from jax.experimental.pallas import tpu as pltpu
from jax.experimental.pallas import tpu_sc as plsc

NC = 2
NS = 16
LANES = 16
CHUNK = 128
NBUF = 4
ROWS = 5000
EPS = 1e-5


def _sc_degree(edges3, zeros8, ones8, n_acc, e_work):
    e_per_tile = e_work // (NC * NS)
    n_rounds = e_per_tile // (CHUNK * NBUF)
    cpt = e_per_tile // CHUNK
    nrt = n_acc // NS
    mesh = plsc.VectorSubcoreMesh(core_axis_name="c", subcore_axis_name="s")

    @functools.partial(
        pl.kernel,
        out_type=jax.ShapeDtypeStruct((NC, n_acc, 8), jnp.float32),
        mesh=mesh,
        compiler_params=pltpu.CompilerParams(use_tc_tiling_on_sc=False),
        scratch_types=[
            pltpu.VMEM((2, NBUF, CHUNK), jnp.int32),
            pltpu.VMEM((CHUNK, 8), jnp.float32),
            pltpu.VMEM_SHARED((n_acc, 8), jnp.float32),
        ] + [pltpu.SemaphoreType.DMA] * NBUF,
    )
    def deg_kernel(edges_hbm, zeros_hbm, ones_hbm, out_hbm, didx, ones_v,
                   acc, *ssem):
        c = lax.axis_index("c")
        s = lax.axis_index("s")
        pltpu.sync_copy(zeros_hbm.at[pl.ds(s * nrt, nrt)],
                        acc.at[pl.ds(s * nrt, nrt)])
        pltpu.sync_copy(ones_hbm, ones_v)
        plsc.subcore_barrier()
        cb = (c * NS + s) * cpt

        def idx_load(r, p):
            pltpu.sync_copy(edges_hbm.at[1, pl.ds(cb + r * NBUF, NBUF)],
                            didx.at[p])

        def scatter_start(p, b):
            pltpu.async_copy(ones_v, acc.at[didx.at[p, b]], ssem[b],
                             add=True)

        def scatter_wait(p, b):
            pltpu.make_async_copy(ones_v, acc.at[didx.at[p, b]],
                                  ssem[b]).wait()

        def round_body(r, p):
            for b in range(NBUF):
                scatter_start(p, b)
            idx_load(r + 1, 1 - p)
            for b in range(NBUF):
                scatter_wait(p, b)

        idx_load(0, 0)
        n_even = n_rounds - (n_rounds % 2)

        @pl.loop(0, n_even, step=2)
        def _(r0):
            round_body(r0, 0)
            round_body(r0 + 1, 1)

        if n_rounds % 2:
            round_body(n_rounds - 1, 0)

        plsc.subcore_barrier()
        pltpu.sync_copy(acc.at[pl.ds(s * nrt, nrt)],
                        out_hbm.at[c, pl.ds(s * nrt, nrt)])

    return deg_kernel(edges3, zeros8, ones8)


def _sc_aggregate(xs_flat, edges3, n_acc, half, e_work):
    e_per_tile = e_work // NS
    n_rounds = e_per_tile // (CHUNK * NBUF)
    cpt = e_per_tile // CHUNK
    nrt = n_acc // NS
    mesh = plsc.VectorSubcoreMesh(core_axis_name="c", subcore_axis_name="s")

    @functools.partial(
        pl.kernel,
        out_type=jax.ShapeDtypeStruct((NC, n_acc, half), jnp.float32),
        mesh=mesh,
        compiler_params=pltpu.CompilerParams(use_tc_tiling_on_sc=False),
        scratch_types=[
            pltpu.VMEM((2, 2, NBUF, CHUNK), jnp.int32),
            pltpu.VMEM((NBUF, CHUNK, half), jnp.float32),
            pltpu.VMEM_SHARED((n_acc, half), jnp.float32),
        ] + [pltpu.SemaphoreType.DMA] * (2 * NBUF),
    )
    def agg_kernel(xs_hbm, edges_hbm, out_hbm, eidx, rows, acc, *sems):
        gsem, ssem = sems[:NBUF], sems[NBUF:]
        c = lax.axis_index("c")
        s = lax.axis_index("s")
        coff = c * n_acc
        pltpu.sync_copy(xs_hbm.at[pl.ds(coff + s * nrt, nrt)],
                        acc.at[pl.ds(s * nrt, nrt)])
        plsc.subcore_barrier()
        cb = s * cpt

        def idx_load(r, p):
            pltpu.sync_copy(edges_hbm.at[:, pl.ds(cb + r * NBUF, NBUF)],
                            eidx.at[p])
            for b in range(NBUF):
                for v in range(CHUNK // LANES):
                    sl = pl.ds(v * LANES, LANES)
                    eidx[p, 0, b, sl] = eidx[p, 0, b, sl] + coff

        def gather_start(p, b):
            pltpu.async_copy(xs_hbm.at[eidx.at[p, 0, b]], rows.at[b],
                             gsem[b])

        def gather_wait(p, b):
            pltpu.make_async_copy(xs_hbm.at[eidx.at[p, 0, b]], rows.at[b],
                                  gsem[b]).wait()

        def scatter_start(p, b):
            pltpu.async_copy(rows.at[b], acc.at[eidx.at[p, 1, b]], ssem[b],
                             add=True)

        def scatter_wait(p, b):
            pltpu.make_async_copy(rows.at[b], acc.at[eidx.at[p, 1, b]],
                                  ssem[b]).wait()

        def round_body(r, p):
            for b in range(NBUF):
                gather_wait(p, b)
                scatter_start(p, b)
            idx_load(r + 1, 1 - p)
            for b in range(NBUF):
                scatter_wait(p, b)
                gather_start(1 - p, b)

        idx_load(0, 0)
        for b in range(NBUF):
            gather_start(0, b)
        n_even = n_rounds - (n_rounds % 2)

        @pl.loop(0, n_even, step=2)
        def _(r0):
            round_body(r0, 0)
            round_body(r0 + 1, 1)

        if n_rounds % 2:
            round_body(n_rounds - 1, 0)

        for b in range(NBUF):
            gather_wait(n_rounds % 2, b)

        plsc.subcore_barrier()
        pltpu.sync_copy(acc.at[pl.ds(s * nrt, nrt)],
                        out_hbm.at[c, pl.ds(s * nrt, nrt)])

    return agg_kernel(xs_flat, edges3)


def _prep_body(x_ref, w_ref, degp_ref, xs_ref, dis_ref):
    degp = degp_ref[...]
    deg = degp[0, :, 0] + degp[1, :, 0] + 1.0
    dis = lax.rsqrt(deg)
    xw = jnp.dot(x_ref[...], w_ref[...], preferred_element_type=jnp.float32)
    xs = xw * dis[:, None]
    half = xs.shape[1] // 2
    xs_ref[0] = xs[:, :half]
    xs_ref[1] = xs[:, half:]
    dis_ref[...] = dis[:, None]


def _layer_mm_body(n_rows, acc_ref, dis_ref, b_ref, g_ref, be_ref, w_ref,
                   out_ref, ssum):
    ph = pl.program_id(0)
    i = pl.program_id(1)
    dis = dis_ref[...][:, 0]
    z = acc_ref[...] * dis[None, :, None] + b_ref[...][:, None, :]

    @pl.when(jnp.logical_and(ph == 0, i == 0))
    def _():
        ssum[...] = jnp.zeros_like(ssum)

    @pl.when(ph == 0)
    def _():
        ssum[0] += jnp.sum(z, axis=1)
        ssum[1] += jnp.sum(z * z, axis=1)

    @pl.when(ph == 1)
    def _():
        st = ssum[...]
        mean = st[0] / n_rows
        var = st[1] / n_rows - mean * mean
        scale = lax.rsqrt(var + EPS) * g_ref[...]
        shift = be_ref[...] - mean * scale
        h = jnp.maximum(z * scale[:, None, :] + shift[:, None, :], 0.0)
        hf = jnp.concatenate([h[0], h[1]], axis=1)
        xw = jnp.dot(hf, w_ref[...], preferred_element_type=jnp.float32)
        xs = xw * dis[:, None]
        half = xw.shape[1] // 2
        out_ref[0] = xs[:, :half]
        out_ref[1] = xs[:, half:]


def _layer_body(n_rows, acc_ref, dis_ref, b_ref, g_ref, be_ref, out_ref,
                ssum):
    ph = pl.program_id(0)
    i = pl.program_id(1)
    dis = dis_ref[...][:, 0]
    z = acc_ref[...] * dis[None, :, None] + b_ref[...][:, None, :]

    @pl.when(jnp.logical_and(ph == 0, i == 0))
    def _():
        ssum[...] = jnp.zeros_like(ssum)

    @pl.when(ph == 0)
    def _():
        ssum[0] += jnp.sum(z, axis=1)
        ssum[1] += jnp.sum(z * z, axis=1)

    @pl.when(ph == 1)
    def _():
        st = ssum[...]
        mean = st[0] / n_rows
        var = st[1] / n_rows - mean * mean
        scale = lax.rsqrt(var + EPS) * g_ref[...]
        shift = be_ref[...] - mean * scale
        h = jnp.maximum(z * scale[:, None, :] + shift[:, None, :], 0.0)
        out_ref[...] = h * dis[None, :, None]


def _final_body(acc_ref, dis_ref, wmu_ref, bmu_ref, wls_ref, bls_ref,
                mu_ref, ls_ref):
    dis = dis_ref[...]
    a = acc_ref[...]
    t = jnp.concatenate([a[0], a[1]], axis=1) * dis
    mu_ref[...] = (jnp.dot(t, wmu_ref[...], preferred_element_type=jnp.float32)
                   + bmu_ref[...])
    ls_ref[...] = (jnp.dot(t, wls_ref[...], preferred_element_type=jnp.float32)
                   + bls_ref[...])


def _full_spec(shape, ndim_grid=1):
    zeros = (0,) * len(shape)
    if ndim_grid == 2:
        return pl.BlockSpec(shape, lambda p, i: zeros)
    return pl.BlockSpec(shape, lambda i: zeros)


def _row_spec(nd_shape, ndim_grid=1):
    if len(nd_shape) == 3:
        if ndim_grid == 2:
            return pl.BlockSpec((nd_shape[0], ROWS, nd_shape[2]),
                                lambda p, i: (0, i, 0))
        return pl.BlockSpec((nd_shape[0], ROWS, nd_shape[2]),
                            lambda i: (0, i, 0))
    if ndim_grid == 2:
        return pl.BlockSpec((ROWS, nd_shape[1]), lambda p, i: (i, 0))
    return pl.BlockSpec((ROWS, nd_shape[1]), lambda i: (i, 0))


def kernel(x, edge_index, W1, b1, g1, be1, W2, b2, g2, be2, Wmu, bmu, Wls,
           bls):
    n, in_ch = x.shape
    hid = W1.shape[1]
    lat = Wmu.shape[1]
    half = hid // 2
    e = edge_index.shape[1]
    idt = edge_index.dtype

    n_blocks = n // ROWS
    nrt = -(-(n + 1) // (NS * 8)) * 8
    n_acc = nrt * NS

    e_unit = NC * NS * CHUNK
    e_work = -(-e // e_unit) * e_unit
    pad = e_work + NBUF * CHUNK - e
    trash = n + jnp.arange(pad, dtype=idt) % (n_acc - n)
    pad_col = jnp.concatenate(
        [jnp.zeros((1, pad), idt), trash[None, :]])
    edges3 = jnp.concatenate([edge_index, pad_col], axis=1).reshape(
        2, -1, CHUNK)

    degp = _sc_degree(edges3, jnp.zeros((n_acc, 8), jnp.float32),
                      jnp.ones((CHUNK, 8), jnp.float32), n_acc, e_work)

    xs1, dis = pl.pallas_call(
        _prep_body,
        grid=(n_blocks,),
        in_specs=[_row_spec((n, in_ch)), _full_spec(W1.shape),
                  _row_spec((NC, n_acc, 8))],
        out_specs=[_row_spec((NC, n_acc, half)), _row_spec((n_acc, 1))],
        out_shape=[jax.ShapeDtypeStruct((NC, n_acc, half), jnp.float32),
                   jax.ShapeDtypeStruct((n_acc, 1), jnp.float32)],
    )(x, W1, degp)

    nf = float(n)

    acc1 = _sc_aggregate(xs1.reshape(NC * n_acc, half), edges3, n_acc, half,
                         e_work)
    xs2 = pl.pallas_call(
        functools.partial(_layer_mm_body, nf),
        grid=(2, n_blocks),
        in_specs=[_row_spec((NC, n_acc, half), 2), _row_spec((n_acc, 1), 2),
                  _full_spec((NC, half), 2), _full_spec((NC, half), 2),
                  _full_spec((NC, half), 2), _full_spec(W2.shape, 2)],
        out_specs=_row_spec((NC, n_acc, half), 2),
        out_shape=jax.ShapeDtypeStruct((NC, n_acc, half), jnp.float32),
        scratch_shapes=[pltpu.VMEM((2, NC, half), jnp.float32)],
    )(acc1, dis, b1.reshape(NC, half), g1.reshape(NC, half),
      be1.reshape(NC, half), W2)

    acc2 = _sc_aggregate(xs2.reshape(NC * n_acc, half), edges3, n_acc, half,
                         e_work)
    xs3 = pl.pallas_call(
        functools.partial(_layer_body, nf),
        grid=(2, n_blocks),
        in_specs=[_row_spec((NC, n_acc, half), 2), _row_spec((n_acc, 1), 2),
                  _full_spec((NC, half), 2), _full_spec((NC, half), 2),
                  _full_spec((NC, half), 2)],
        out_specs=_row_spec((NC, n_acc, half), 2),
        out_shape=jax.ShapeDtypeStruct((NC, n_acc, half), jnp.float32),
        scratch_shapes=[pltpu.VMEM((2, NC, half), jnp.float32)],
    )(acc2, dis, b2.reshape(NC, half), g2.reshape(NC, half),
      be2.reshape(NC, half))

    acc3 = _sc_aggregate(xs3.reshape(NC * n_acc, half), edges3, n_acc, half,
                         e_work)
    mu, ls = pl.pallas_call(
        _final_body,
        grid=(n_blocks,),
        in_specs=[_row_spec((NC, n_acc, half)), _row_spec((n_acc, 1)),
                  _full_spec(Wmu.shape), _full_spec((1, lat)),
                  _full_spec(Wls.shape), _full_spec((1, lat))],
        out_specs=[_row_spec((n, lat)), _row_spec((n, lat))],
        out_shape=[jax.ShapeDtypeStruct((n, lat), jnp.float32),
                   jax.ShapeDtypeStruct((n, lat), jnp.float32)],
    )(acc3, dis, Wmu, bmu.reshape(1, lat), Wls, bls.reshape(1, lat))

    return (mu, ls)


# speedup vs baseline: 1.5445x; 1.0420x over previous
import functools

import jax
import jax.numpy as jnp
from jax import lax
from jax.experimental import pallas as pl
from jax.experimental.pallas import tpu as pltpu
from jax.experimental.pallas import tpu_sc as plsc

NC = 2
NS = 16
LANES = 16
CHUNK = 128
NBUF = 5
ROWS = 5000
EPS = 1e-5


def _sc_degree(edges3, zeros8, ones8, n_acc, e_work):
    e_per_tile = e_work // (NC * NS)
    n_rounds = e_per_tile // (CHUNK * NBUF)
    cpt = e_per_tile // CHUNK
    nrt = n_acc // NS
    mesh = plsc.VectorSubcoreMesh(core_axis_name="c", subcore_axis_name="s")

    @functools.partial(
        pl.kernel,
        out_type=jax.ShapeDtypeStruct((NC, n_acc, 8), jnp.float32),
        mesh=mesh,
        compiler_params=pltpu.CompilerParams(use_tc_tiling_on_sc=False),
        scratch_types=[
            pltpu.VMEM((2, NBUF, CHUNK), jnp.int32),
            pltpu.VMEM((CHUNK, 8), jnp.float32),
            pltpu.VMEM_SHARED((n_acc, 8), jnp.float32),
        ] + [pltpu.SemaphoreType.DMA] * NBUF,
    )
    def deg_kernel(edges_hbm, zeros_hbm, ones_hbm, out_hbm, didx, ones_v,
                   acc, *ssem):
        c = lax.axis_index("c")
        s = lax.axis_index("s")
        pltpu.sync_copy(zeros_hbm.at[pl.ds(s * nrt, nrt)],
                        acc.at[pl.ds(s * nrt, nrt)])
        pltpu.sync_copy(ones_hbm, ones_v)
        plsc.subcore_barrier()
        cb = (c * NS + s) * cpt

        def idx_load(r, p):
            pltpu.sync_copy(edges_hbm.at[1, pl.ds(cb + r * NBUF, NBUF)],
                            didx.at[p])

        def scatter_start(p, b):
            pltpu.async_copy(ones_v, acc.at[didx.at[p, b]], ssem[b],
                             add=True)

        def scatter_wait(p, b):
            pltpu.make_async_copy(ones_v, acc.at[didx.at[p, b]],
                                  ssem[b]).wait()

        def round_body(r, p):
            for b in range(NBUF):
                scatter_start(p, b)
            idx_load(r + 1, 1 - p)
            for b in range(NBUF):
                scatter_wait(p, b)

        idx_load(0, 0)
        n_even = n_rounds - (n_rounds % 2)

        @pl.loop(0, n_even, step=2)
        def _(r0):
            round_body(r0, 0)
            round_body(r0 + 1, 1)

        if n_rounds % 2:
            round_body(n_rounds - 1, 0)

        plsc.subcore_barrier()
        pltpu.sync_copy(acc.at[pl.ds(s * nrt, nrt)],
                        out_hbm.at[c, pl.ds(s * nrt, nrt)])

    return deg_kernel(edges3, zeros8, ones8)


def _sc_aggregate(xs_flat, edges3, n_acc, half, e_work):
    e_per_tile = e_work // NS
    n_rounds = e_per_tile // (CHUNK * NBUF)
    cpt = e_per_tile // CHUNK
    nrt = n_acc // NS
    mesh = plsc.VectorSubcoreMesh(core_axis_name="c", subcore_axis_name="s")

    @functools.partial(
        pl.kernel,
        out_type=jax.ShapeDtypeStruct((NC, n_acc, half), jnp.float32),
        mesh=mesh,
        compiler_params=pltpu.CompilerParams(use_tc_tiling_on_sc=False),
        scratch_types=[
            pltpu.VMEM((2, 2, NBUF, CHUNK), jnp.int32),
            pltpu.VMEM((NBUF, CHUNK, half), jnp.float32),
            pltpu.VMEM_SHARED((n_acc, half), jnp.float32),
        ] + [pltpu.SemaphoreType.DMA] * (2 * NBUF),
    )
    def agg_kernel(xs_hbm, edges_hbm, out_hbm, eidx, rows, acc, *sems):
        gsem, ssem = sems[:NBUF], sems[NBUF:]
        c = lax.axis_index("c")
        s = lax.axis_index("s")
        coff = c * n_acc
        pltpu.sync_copy(xs_hbm.at[pl.ds(coff + s * nrt, nrt)],
                        acc.at[pl.ds(s * nrt, nrt)])
        plsc.subcore_barrier()
        cb = s * cpt

        def idx_load(r, p):
            pltpu.sync_copy(edges_hbm.at[:, pl.ds(cb + r * NBUF, NBUF)],
                            eidx.at[p])
            for b in range(NBUF):
                for v in range(CHUNK // LANES):
                    sl = pl.ds(v * LANES, LANES)
                    eidx[p, 0, b, sl] = eidx[p, 0, b, sl] + coff

        def gather_start(p, b):
            pltpu.async_copy(xs_hbm.at[eidx.at[p, 0, b]], rows.at[b],
                             gsem[b])

        def gather_wait(p, b):
            pltpu.make_async_copy(xs_hbm.at[eidx.at[p, 0, b]], rows.at[b],
                                  gsem[b]).wait()

        def scatter_start(p, b):
            pltpu.async_copy(rows.at[b], acc.at[eidx.at[p, 1, b]], ssem[b],
                             add=True)

        def scatter_wait(p, b):
            pltpu.make_async_copy(rows.at[b], acc.at[eidx.at[p, 1, b]],
                                  ssem[b]).wait()

        def round_body(r, p):
            for b in range(NBUF):
                gather_wait(p, b)
                scatter_start(p, b)
            idx_load(r + 1, 1 - p)
            for b in range(NBUF):
                scatter_wait(p, b)
                gather_start(1 - p, b)

        idx_load(0, 0)
        for b in range(NBUF):
            gather_start(0, b)
        n_even = n_rounds - (n_rounds % 2)

        @pl.loop(0, n_even, step=2)
        def _(r0):
            round_body(r0, 0)
            round_body(r0 + 1, 1)

        if n_rounds % 2:
            round_body(n_rounds - 1, 0)

        for b in range(NBUF):
            gather_wait(n_rounds % 2, b)

        plsc.subcore_barrier()
        pltpu.sync_copy(acc.at[pl.ds(s * nrt, nrt)],
                        out_hbm.at[c, pl.ds(s * nrt, nrt)])

    return agg_kernel(xs_flat, edges3)


def _mm_body(x_ref, w_ref, xw_ref):
    xw_ref[...] = jnp.dot(x_ref[...], w_ref[...],
                          preferred_element_type=jnp.float32)


def _prep_body(xw_ref, degp_ref, xs_ref, dis_ref):
    degp = degp_ref[...]
    deg = degp[0, :, 0] + degp[1, :, 0] + 1.0
    dis = lax.rsqrt(deg)
    xs = xw_ref[...] * dis[:, None]
    half = xs.shape[1] // 2
    xs_ref[0] = xs[:, :half]
    xs_ref[1] = xs[:, half:]
    dis_ref[...] = dis[:, None]


def _layer_mm_body(n_rows, acc_ref, dis_ref, b_ref, g_ref, be_ref, w_ref,
                   out_ref, ssum):
    ph = pl.program_id(0)
    i = pl.program_id(1)
    dis = dis_ref[...][:, 0]
    z = acc_ref[...] * dis[None, :, None] + b_ref[...][:, None, :]

    @pl.when(jnp.logical_and(ph == 0, i == 0))
    def _():
        ssum[...] = jnp.zeros_like(ssum)

    @pl.when(ph == 0)
    def _():
        ssum[0] += jnp.sum(z, axis=1)
        ssum[1] += jnp.sum(z * z, axis=1)

    @pl.when(ph == 1)
    def _():
        st = ssum[...]
        mean = st[0] / n_rows
        var = st[1] / n_rows - mean * mean
        scale = lax.rsqrt(var + EPS) * g_ref[...]
        shift = be_ref[...] - mean * scale
        h = jnp.maximum(z * scale[:, None, :] + shift[:, None, :], 0.0)
        hf = jnp.concatenate([h[0], h[1]], axis=1)
        xw = jnp.dot(hf, w_ref[...], preferred_element_type=jnp.float32)
        xs = xw * dis[:, None]
        half = xw.shape[1] // 2
        out_ref[0] = xs[:, :half]
        out_ref[1] = xs[:, half:]


def _layer_body(n_rows, acc_ref, dis_ref, b_ref, g_ref, be_ref, out_ref,
                ssum):
    ph = pl.program_id(0)
    i = pl.program_id(1)
    dis = dis_ref[...][:, 0]
    z = acc_ref[...] * dis[None, :, None] + b_ref[...][:, None, :]

    @pl.when(jnp.logical_and(ph == 0, i == 0))
    def _():
        ssum[...] = jnp.zeros_like(ssum)

    @pl.when(ph == 0)
    def _():
        ssum[0] += jnp.sum(z, axis=1)
        ssum[1] += jnp.sum(z * z, axis=1)

    @pl.when(ph == 1)
    def _():
        st = ssum[...]
        mean = st[0] / n_rows
        var = st[1] / n_rows - mean * mean
        scale = lax.rsqrt(var + EPS) * g_ref[...]
        shift = be_ref[...] - mean * scale
        h = jnp.maximum(z * scale[:, None, :] + shift[:, None, :], 0.0)
        out_ref[...] = h * dis[None, :, None]


def _final_body(acc_ref, dis_ref, wmu_ref, bmu_ref, wls_ref, bls_ref,
                mu_ref, ls_ref):
    dis = dis_ref[...]
    a = acc_ref[...]
    t = jnp.concatenate([a[0], a[1]], axis=1) * dis
    mu_ref[...] = (jnp.dot(t, wmu_ref[...], preferred_element_type=jnp.float32)
                   + bmu_ref[...])
    ls_ref[...] = (jnp.dot(t, wls_ref[...], preferred_element_type=jnp.float32)
                   + bls_ref[...])


def _full_spec(shape, ndim_grid=1):
    zeros = (0,) * len(shape)
    if ndim_grid == 2:
        return pl.BlockSpec(shape, lambda p, i: zeros)
    return pl.BlockSpec(shape, lambda i: zeros)


def _row_spec(nd_shape, ndim_grid=1):
    if len(nd_shape) == 3:
        if ndim_grid == 2:
            return pl.BlockSpec((nd_shape[0], ROWS, nd_shape[2]),
                                lambda p, i: (0, i, 0))
        return pl.BlockSpec((nd_shape[0], ROWS, nd_shape[2]),
                            lambda i: (0, i, 0))
    if ndim_grid == 2:
        return pl.BlockSpec((ROWS, nd_shape[1]), lambda p, i: (i, 0))
    return pl.BlockSpec((ROWS, nd_shape[1]), lambda i: (i, 0))


def kernel(x, edge_index, W1, b1, g1, be1, W2, b2, g2, be2, Wmu, bmu, Wls,
           bls):
    n, in_ch = x.shape
    hid = W1.shape[1]
    lat = Wmu.shape[1]
    half = hid // 2
    e = edge_index.shape[1]
    idt = edge_index.dtype

    n_blocks = n // ROWS
    nrt = -(-(n + 1) // (NS * 8)) * 8
    n_acc = nrt * NS

    e_unit = NC * NS * CHUNK
    e_work = -(-e // e_unit) * e_unit
    pad = e_work + NBUF * CHUNK - e
    trash = n + jnp.arange(pad, dtype=idt) % (n_acc - n)
    pad_col = jnp.concatenate(
        [jnp.zeros((1, pad), idt), trash[None, :]])
    edges3 = jnp.concatenate([edge_index, pad_col], axis=1).reshape(
        2, -1, CHUNK)

    degp = _sc_degree(edges3, jnp.zeros((n_acc, 8), jnp.float32),
                      jnp.ones((CHUNK, 8), jnp.float32), n_acc, e_work)

    xw1 = pl.pallas_call(
        _mm_body,
        grid=(n_blocks,),
        in_specs=[_row_spec((n, in_ch)), _full_spec(W1.shape)],
        out_specs=_row_spec((n, hid)),
        out_shape=jax.ShapeDtypeStruct((n, hid), jnp.float32),
    )(x, W1)

    xs1, dis = pl.pallas_call(
        _prep_body,
        grid=(n_blocks,),
        in_specs=[_row_spec((n, hid)), _row_spec((NC, n_acc, 8))],
        out_specs=[_row_spec((NC, n_acc, half)), _row_spec((n_acc, 1))],
        out_shape=[jax.ShapeDtypeStruct((NC, n_acc, half), jnp.float32),
                   jax.ShapeDtypeStruct((n_acc, 1), jnp.float32)],
    )(xw1, degp)

    nf = float(n)

    acc1 = _sc_aggregate(xs1.reshape(NC * n_acc, half), edges3, n_acc, half,
                         e_work)
    xs2 = pl.pallas_call(
        functools.partial(_layer_mm_body, nf),
        grid=(2, n_blocks),
        in_specs=[_row_spec((NC, n_acc, half), 2), _row_spec((n_acc, 1), 2),
                  _full_spec((NC, half), 2), _full_spec((NC, half), 2),
                  _full_spec((NC, half), 2), _full_spec(W2.shape, 2)],
        out_specs=_row_spec((NC, n_acc, half), 2),
        out_shape=jax.ShapeDtypeStruct((NC, n_acc, half), jnp.float32),
        scratch_shapes=[pltpu.VMEM((2, NC, half), jnp.float32)],
    )(acc1, dis, b1.reshape(NC, half), g1.reshape(NC, half),
      be1.reshape(NC, half), W2)

    acc2 = _sc_aggregate(xs2.reshape(NC * n_acc, half), edges3, n_acc, half,
                         e_work)
    xs3 = pl.pallas_call(
        functools.partial(_layer_body, nf),
        grid=(2, n_blocks),
        in_specs=[_row_spec((NC, n_acc, half), 2), _row_spec((n_acc, 1), 2),
                  _full_spec((NC, half), 2), _full_spec((NC, half), 2),
                  _full_spec((NC, half), 2)],
        out_specs=_row_spec((NC, n_acc, half), 2),
        out_shape=jax.ShapeDtypeStruct((NC, n_acc, half), jnp.float32),
        scratch_shapes=[pltpu.VMEM((2, NC, half), jnp.float32)],
    )(acc2, dis, b2.reshape(NC, half), g2.reshape(NC, half),
      be2.reshape(NC, half))

    acc3 = _sc_aggregate(xs3.reshape(NC * n_acc, half), edges3, n_acc, half,
                         e_work)
    mu, ls = pl.pallas_call(
        _final_body,
        grid=(n_blocks,),
        in_specs=[_row_spec((NC, n_acc, half)), _row_spec((n_acc, 1)),
                  _full_spec(Wmu.shape), _full_spec((1, lat)),
                  _full_spec(Wls.shape), _full_spec((1, lat))],
        out_specs=[_row_spec((n, lat)), _row_spec((n, lat))],
        out_shape=[jax.ShapeDtypeStruct((n, lat), jnp.float32),
                   jax.ShapeDtypeStruct((n, lat), jnp.float32)],
    )(acc3, dis, Wmu, bmu.reshape(1, lat), Wls, bls.reshape(1, lat))

    return (mu, ls)
